# baseline (speedup 1.0000x reference)
import functools

import jax
import jax.numpy as jnp
from jax import lax
from jax.experimental import pallas as pl
from jax.experimental.pallas import tpu as pltpu
from jax.experimental.pallas import tpu_sc as plsc

V = 100000
EMB = 64
HID = 128
CD = 32
K = 1024
B = 4
L = 256
N = B * L

TV = 2048


def _sc_gather(table, idx):
    info = plsc.get_sparse_core_info()
    nw = info.num_cores * info.num_subcores
    b_per_w = N // nw
    mesh = plsc.VectorSubcoreMesh(core_axis_name="c", subcore_axis_name="s")

    @functools.partial(
        pl.kernel,
        mesh=mesh,
        out_type=jax.ShapeDtypeStruct((N, EMB), jnp.float32),
        scratch_types=[
            pltpu.VMEM((b_per_w,), jnp.int32),
            pltpu.VMEM((b_per_w, EMB), jnp.float32),
            pltpu.SemaphoreType.DMA,
        ],
        compiler_params=pltpu.CompilerParams(use_tc_tiling_on_sc=False),
    )
    def gather_kernel(table_hbm, idx_hbm, out_hbm, idx_v, rows_v, sem):
        wid = lax.axis_index("s") * info.num_cores + lax.axis_index("c")
        base = wid * b_per_w
        pltpu.sync_copy(idx_hbm.at[pl.ds(base, b_per_w)], idx_v)
        pltpu.async_copy(table_hbm.at[idx_v], rows_v, sem).wait()
        pltpu.sync_copy(rows_v, out_hbm.at[pl.ds(base, b_per_w)])

    return gather_kernel(table, idx)


def _shifted(x, direction, row):
    c = x.shape[1]
    zero = jnp.zeros((1, c), jnp.float32)
    if direction == -1:
        sh = jnp.concatenate([zero, x[:-1, :]], axis=0)
        mask = (row % L) != 0
    else:
        sh = jnp.concatenate([x[1:, :], zero], axis=0)
        mask = (row % L) != (L - 1)
    return jnp.where(mask, sh, 0.0)


def _mm(a, b):
    return lax.dot_general(a, b, (((1,), (0,)), ((), ())),
                           preferred_element_type=jnp.float32)


def _mm_t(a, b):
    return lax.dot_general(a, b, (((1,), (1,)), ((), ())),
                           preferred_element_type=jnp.float32)


def _conv3(x, w0, w1, w2, bias, row):
    y = _mm(x, w1)
    y += _mm(_shifted(x, -1, row), w0)
    y += _mm(_shifted(x, +1, row), w2)
    return y + bias


def _layer_norm(x, g, b, eps=1e-5):
    m = jnp.mean(x, axis=-1, keepdims=True)
    v = jnp.mean((x - m) ** 2, axis=-1, keepdims=True)
    return (x - m) / jnp.sqrt(v + eps) * g + b


def _prefix_body(emb_ref,
                 e1w0, e1w1, e1w2, e1b,
                 e2w0, e2w1, e2w2, e2b,
                 e3w, e3b, elng, elnb,
                 cb_ref,
                 d1w, d1b,
                 d2w0, d2w1, d2w2, d2b,
                 d3w, d3b, dlng, dlnb,
                 h_out, codes_out, loss_out):
    row = lax.broadcasted_iota(jnp.int32, (N, 1), 0)

    h = emb_ref[...]
    h = jnp.maximum(_conv3(h, e1w0[...], e1w1[...], e1w2[...], e1b[...], row), 0.0)
    h = jnp.maximum(_conv3(h, e2w0[...], e2w1[...], e2w2[...], e2b[...], row), 0.0)
    h = _mm(h, e3w[...]) + e3b[...]
    z_e = _layer_norm(h, elng[...], elnb[...])

    cb = cb_ref[...]
    cb_sq_row = _mm_t(jnp.ones((1, CD), jnp.float32), cb * cb)
    scores = cb_sq_row - 2.0 * _mm_t(z_e, cb)
    mins = jnp.min(scores, axis=1, keepdims=True)
    col = lax.broadcasted_iota(jnp.int32, (N, K), 1)
    codes = jnp.min(jnp.where(scores == mins, col, K), axis=1, keepdims=True)
    codes_out[...] = codes

    one_hot = (col == codes).astype(jnp.float32)
    z_q = _mm(one_hot, cb)

    loss_out[0, 0] = 0.1 * jnp.mean((z_e - z_q) ** 2)

    h = jnp.maximum(_mm(z_q, d1w[...]) + d1b[...], 0.0)
    h = jnp.maximum(_conv3(h, d2w0[...], d2w1[...], d2w2[...], d2b[...], row), 0.0)
    h = jnp.maximum(_mm(h, d3w[...]) + d3b[...], 0.0)
    h_out[...] = _layer_norm(h, dlng[...], dlnb[...])


def _prefix_call(emb, enc, cb, dec, interpret=False):
    out_shapes = (
        jax.ShapeDtypeStruct((N, EMB), jnp.float32),
        jax.ShapeDtypeStruct((N, 1), jnp.int32),
        jax.ShapeDtypeStruct((1, 1), jnp.float32),
    )
    n_in = 1 + len(enc) + 1 + len(dec)
    in_specs = [pl.BlockSpec(memory_space=pltpu.VMEM) for _ in range(n_in)]
    out_specs = (
        pl.BlockSpec(memory_space=pltpu.VMEM),
        pl.BlockSpec(memory_space=pltpu.VMEM),
        pl.BlockSpec(memory_space=pltpu.SMEM),
    )
    return pl.pallas_call(
        _prefix_body,
        out_shape=out_shapes,
        in_specs=in_specs,
        out_specs=out_specs,
        interpret=interpret,
    )(emb, *enc, cb, *dec)


def _proj_body(h_ref, w_ref, b_ref, o_ref):
    o_ref[...] = _mm_t(h_ref[...], w_ref[...]) + b_ref[...]


def _proj_call(h, out_w, out_b_row, interpret=False):
    grid = (pl.cdiv(V, TV),)
    return pl.pallas_call(
        _proj_body,
        grid=grid,
        in_specs=[
            pl.BlockSpec((N, EMB), lambda i: (0, 0)),
            pl.BlockSpec((TV, EMB), lambda i: (i, 0)),
            pl.BlockSpec((1, TV), lambda i: (0, i)),
        ],
        out_specs=pl.BlockSpec((N, TV), lambda i: (0, i)),
        out_shape=jax.ShapeDtypeStruct((N, V), jnp.float32),
        compiler_params=pltpu.CompilerParams(
            dimension_semantics=("arbitrary",),
        ),
        interpret=interpret,
    )(h, out_w, out_b_row)


def kernel(x, tok_emb, enc_w1, enc_b1, enc_w2, enc_b2, enc_w3, enc_b3,
           enc_ln_g, enc_ln_b, codebook, dec_w1, dec_b1, dec_w2, dec_b2,
           dec_w3, dec_b3, dec_ln_g, dec_ln_b, out_w, out_b):
    idx = x.reshape(-1).astype(jnp.int32)
    emb = _sc_gather(tok_emb, idx)

    enc = (
        enc_w1[:, :, 0].T, enc_w1[:, :, 1].T, enc_w1[:, :, 2].T,
        enc_b1[None, :],
        enc_w2[:, :, 0].T, enc_w2[:, :, 1].T, enc_w2[:, :, 2].T,
        enc_b2[None, :],
        enc_w3[:, :, 0].T, enc_b3[None, :],
        enc_ln_g[None, :], enc_ln_b[None, :],
    )
    dec = (
        dec_w1[:, :, 0].T, dec_b1[None, :],
        dec_w2[:, :, 0].T, dec_w2[:, :, 1].T, dec_w2[:, :, 2].T,
        dec_b2[None, :],
        dec_w3[:, :, 0].T, dec_b3[None, :],
        dec_ln_g[None, :], dec_ln_b[None, :],
    )
    h_final, codes, loss = _prefix_call(emb, enc, codebook, dec)

    logits = _proj_call(h_final, out_w, out_b[None, :])

    return (logits.reshape(B, L, V), loss[0, 0], codes.reshape(B, L))


# speedup vs baseline: 1.0014x; 1.0014x over previous
import functools

import jax
import jax.numpy as jnp
from jax import lax
from jax.experimental import pallas as pl
from jax.experimental.pallas import tpu as pltpu
from jax.experimental.pallas import tpu_sc as plsc

V = 100000
EMB = 64
HID = 128
CD = 32
K = 1024
B = 4
L = 256
N = B * L

TV = 2048


def _sc_gather(table2, idx):
    info = plsc.get_sparse_core_info()
    nw = info.num_cores * info.num_subcores
    b_per_w = N // nw
    mesh = plsc.VectorSubcoreMesh(core_axis_name="c", subcore_axis_name="s")

    @functools.partial(
        pl.kernel,
        mesh=mesh,
        out_type=jax.ShapeDtypeStruct((N, 2 * EMB), jnp.float32),
        scratch_types=[
            pltpu.VMEM((b_per_w,), jnp.int32),
            pltpu.VMEM((b_per_w, 2 * EMB), jnp.float32),
            pltpu.SemaphoreType.DMA,
        ],
    )
    def gather_kernel(table_hbm, idx_hbm, out_hbm, idx_v, rows_v, sem):
        wid = lax.axis_index("s") * info.num_cores + lax.axis_index("c")
        base = wid * b_per_w
        pltpu.sync_copy(idx_hbm.at[pl.ds(base, b_per_w)], idx_v)
        pltpu.async_copy(table_hbm.at[idx_v], rows_v, sem).wait()
        pltpu.sync_copy(rows_v, out_hbm.at[pl.ds(base, b_per_w)])

    return gather_kernel(table2, idx)


def _shifted(x, direction, row):
    c = x.shape[1]
    zero = jnp.zeros((1, c), jnp.float32)
    if direction == -1:
        sh = jnp.concatenate([zero, x[:-1, :]], axis=0)
        mask = (row % L) != 0
    else:
        sh = jnp.concatenate([x[1:, :], zero], axis=0)
        mask = (row % L) != (L - 1)
    return jnp.where(mask, sh, 0.0)


def _mm(a, b):
    return lax.dot_general(a, b, (((1,), (0,)), ((), ())),
                           preferred_element_type=jnp.float32)


def _mm_t(a, b):
    return lax.dot_general(a, b, (((1,), (1,)), ((), ())),
                           preferred_element_type=jnp.float32)


def _conv3(x, w0, w1, w2, bias, row):
    y = _mm(x, w1)
    y += _mm(_shifted(x, -1, row), w0)
    y += _mm(_shifted(x, +1, row), w2)
    return y + bias


def _layer_norm(x, g, b, eps=1e-5):
    m = jnp.mean(x, axis=-1, keepdims=True)
    v = jnp.mean((x - m) ** 2, axis=-1, keepdims=True)
    return (x - m) / jnp.sqrt(v + eps) * g + b


def _prefix_body(rows_ref, parity_ref,
                 e1w0, e1w1, e1w2, e1b,
                 e2w0, e2w1, e2w2, e2b,
                 e3w, e3b, elng, elnb,
                 cb_ref,
                 d1w, d1b,
                 d2w0, d2w1, d2w2, d2b,
                 d3w, d3b, dlng, dlnb,
                 h_out, codes_out, loss_out):
    row = lax.broadcasted_iota(jnp.int32, (N, 1), 0)

    rows = rows_ref[...]
    h = jnp.where(parity_ref[...] == 0, rows[:, :EMB], rows[:, EMB:])
    h = jnp.maximum(_conv3(h, e1w0[...], e1w1[...], e1w2[...], e1b[...], row), 0.0)
    h = jnp.maximum(_conv3(h, e2w0[...], e2w1[...], e2w2[...], e2b[...], row), 0.0)
    h = _mm(h, e3w[...]) + e3b[...]
    z_e = _layer_norm(h, elng[...], elnb[...])

    cb = cb_ref[...]
    cb_sq_row = _mm_t(jnp.ones((1, CD), jnp.float32), cb * cb)
    scores = cb_sq_row - 2.0 * _mm_t(z_e, cb)
    mins = jnp.min(scores, axis=1, keepdims=True)
    col = lax.broadcasted_iota(jnp.int32, (N, K), 1)
    codes = jnp.min(jnp.where(scores == mins, col, K), axis=1, keepdims=True)
    codes_out[...] = codes

    one_hot = (col == codes).astype(jnp.float32)
    z_q = _mm(one_hot, cb)

    loss_out[0, 0] = 0.1 * jnp.mean((z_e - z_q) ** 2)

    h = jnp.maximum(_mm(z_q, d1w[...]) + d1b[...], 0.0)
    h = jnp.maximum(_conv3(h, d2w0[...], d2w1[...], d2w2[...], d2b[...], row), 0.0)
    h = jnp.maximum(_mm(h, d3w[...]) + d3b[...], 0.0)
    h_out[...] = _layer_norm(h, dlng[...], dlnb[...])


def _prefix_call(rows, parity, enc, cb, dec, interpret=False):
    out_shapes = (
        jax.ShapeDtypeStruct((N, EMB), jnp.float32),
        jax.ShapeDtypeStruct((N, 1), jnp.int32),
        jax.ShapeDtypeStruct((1, 1), jnp.float32),
    )
    n_in = 2 + len(enc) + 1 + len(dec)
    in_specs = [pl.BlockSpec(memory_space=pltpu.VMEM) for _ in range(n_in)]
    out_specs = (
        pl.BlockSpec(memory_space=pltpu.VMEM),
        pl.BlockSpec(memory_space=pltpu.VMEM),
        pl.BlockSpec(memory_space=pltpu.SMEM),
    )
    return pl.pallas_call(
        _prefix_body,
        out_shape=out_shapes,
        in_specs=in_specs,
        out_specs=out_specs,
        interpret=interpret,
    )(rows, parity, *enc, cb, *dec)


def _proj_body(h_ref, w_ref, b_ref, o_ref):
    o_ref[...] = _mm_t(h_ref[...], w_ref[...]) + b_ref[...]


def _proj_call(h, out_w, out_b_row, interpret=False):
    grid = (pl.cdiv(V, TV),)
    return pl.pallas_call(
        _proj_body,
        grid=grid,
        in_specs=[
            pl.BlockSpec((N, EMB), lambda i: (0, 0)),
            pl.BlockSpec((TV, EMB), lambda i: (i, 0)),
            pl.BlockSpec((1, TV), lambda i: (0, i)),
        ],
        out_specs=pl.BlockSpec((N, TV), lambda i: (0, i)),
        out_shape=jax.ShapeDtypeStruct((N, V), jnp.float32),
        compiler_params=pltpu.CompilerParams(
            dimension_semantics=("arbitrary",),
        ),
        interpret=interpret,
    )(h, out_w, out_b_row)


def kernel(x, tok_emb, enc_w1, enc_b1, enc_w2, enc_b2, enc_w3, enc_b3,
           enc_ln_g, enc_ln_b, codebook, dec_w1, dec_b1, dec_w2, dec_b2,
           dec_w3, dec_b3, dec_ln_g, dec_ln_b, out_w, out_b):
    idx = x.reshape(-1).astype(jnp.int32)
    table2 = tok_emb.reshape(V // 2, 2 * EMB)
    rows = _sc_gather(table2, idx // 2)
    parity = (idx % 2).reshape(N, 1)

    enc = (
        enc_w1[:, :, 0].T, enc_w1[:, :, 1].T, enc_w1[:, :, 2].T,
        enc_b1[None, :],
        enc_w2[:, :, 0].T, enc_w2[:, :, 1].T, enc_w2[:, :, 2].T,
        enc_b2[None, :],
        enc_w3[:, :, 0].T, enc_b3[None, :],
        enc_ln_g[None, :], enc_ln_b[None, :],
    )
    dec = (
        dec_w1[:, :, 0].T, dec_b1[None, :],
        dec_w2[:, :, 0].T, dec_w2[:, :, 1].T, dec_w2[:, :, 2].T,
        dec_b2[None, :],
        dec_w3[:, :, 0].T, dec_b3[None, :],
        dec_ln_g[None, :], dec_ln_b[None, :],
    )
    h_final, codes, loss = _prefix_call(rows, parity, enc, codebook, dec)

    logits = _proj_call(h_final, out_w, out_b[None, :])

    return (logits.reshape(B, L, V), loss[0, 0], codes.reshape(B, L))
